# baseline (speedup 1.0000x reference)
import functools
import math

import jax
import jax.numpy as jnp
import numpy as np
from jax.experimental import pallas as pl
from jax.experimental.pallas import tpu as pltpu


def _nt_dot(a, b):
    return jax.lax.dot_general(
        a, b, (((1,), (1,)), ((), ())), preferred_element_type=jnp.float32
    )


def _tn_dot_exact(a, b):
    return jax.lax.dot_general(
        a, b, (((0,), (0,)), ((), ())),
        preferred_element_type=jnp.float32,
        precision=jax.lax.Precision.HIGHEST,
    )


def _measure_kernel(q_ref, k_ref, m_ref, *, L, D, HP, bk):
    b = pl.program_id(0)
    hp = pl.program_id(1)
    r0 = 2 * (b * HP + hp)
    for hh in range(2):
        cols = slice(hh * D, (hh + 1) * D)
        q_h = q_ref[0][:, cols]
        m_acc = jnp.full((8, L), -jnp.inf, dtype=jnp.float32)
        s_acc = jnp.zeros((8, L), jnp.float32)
        for jb in range(L // bk):
            k_chunk = k_ref[0, jb * bk:(jb + 1) * bk, cols]
            s_t = _nt_dot(k_chunk, q_h)
            for sb in range(bk // 8):
                v = s_t[sb * 8:(sb + 1) * 8, :]
                m_acc = jnp.maximum(m_acc, v)
                s_acc = s_acc + v
        m_max = jnp.max(m_acc, axis=0, keepdims=True)
        row_sum = jnp.sum(s_acc, axis=0, keepdims=True)
        m_ref[pl.ds(r0 + hh, 1), :] = m_max - row_sum * (1.0 / L)


def _topk_kernel(m_ref, idx_ref, *, L, R, u, u_pad):
    m_cur = m_ref[...]
    idx2d = jax.lax.broadcasted_iota(jnp.int32, (R, L), 1)
    for t in range(u_pad):
        if t < u:
            rmax = jnp.max(m_cur, axis=1, keepdims=True)
            sel = jnp.min(
                jnp.where(m_cur == rmax, idx2d, L), axis=1, keepdims=True
            )
            idx_ref[:, t:t + 1] = sel
            m_cur = jnp.where(idx2d == sel, -jnp.inf, m_cur)
        else:
            idx_ref[:, t:t + 1] = jnp.full((R, 1), -1, jnp.int32)


def _attend_kernel(idx_ref, q_ref, k_ref, v_ref, out_ref, *, L, D, HP, u, u_pad):
    b = pl.program_id(0)
    hp = pl.program_id(1)
    r0 = 2 * (b * HP + hp)
    lane_i = jax.lax.broadcasted_iota(jnp.int32, (u_pad, L), 1)
    scale = 1.0 / math.sqrt(D)
    for hh in range(2):
        cols = slice(hh * D, (hh + 1) * D)
        idx_row = idx_ref[pl.ds(r0 + hh, 1), :]
        idx_col = jnp.transpose(idx_row, (1, 0))
        p_sel = jnp.where(idx_col == lane_i, 1.0, 0.0)
        q_h = q_ref[0][:, cols]
        k_h = k_ref[0][:, cols]
        v_h = v_ref[0][:, cols]
        q_r = jax.lax.dot_general(
            p_sel, q_h, (((1,), (0,)), ((), ())),
            preferred_element_type=jnp.float32,
            precision=jax.lax.Precision.HIGHEST,
        )
        scores = _nt_dot(q_r, k_h) * scale
        s_max = jnp.max(scores, axis=1, keepdims=True)
        s_exp = jnp.exp(scores - s_max)
        attn = s_exp / jnp.sum(s_exp, axis=1, keepdims=True)
        upd = jax.lax.dot_general(
            attn, v_h, (((1,), (0,)), ((), ())),
            preferred_element_type=jnp.float32,
        )
        scat = _tn_dot_exact(p_sel, upd)
        msk = _tn_dot_exact(p_sel, jnp.ones((u_pad, 1), jnp.float32))
        v_mean = jnp.sum(v_h, axis=0, keepdims=True) * (1.0 / L)
        out_ref[0, hh] = jnp.where(
            msk > 0.5, scat, jnp.broadcast_to(v_mean, (L, D))
        )


def kernel(queries, keys, values, attn_mask):
    B, L, H, D = queries.shape
    assert H % 2 == 0
    HP = H // 2
    R = B * H
    u = 5 * int(np.ceil(np.log(L)))
    u = min(u, L)
    u_pad = ((u + 7) // 8) * 8
    bk = min(512, L)

    q2 = jnp.reshape(queries, (B, L, H * D))
    k2 = jnp.reshape(keys, (B, L, H * D))
    v2 = jnp.reshape(values, (B, L, H * D))

    pair = pl.BlockSpec((1, L, 2 * D), lambda b, hp: (b, 0, hp))

    m_all = pl.pallas_call(
        functools.partial(_measure_kernel, L=L, D=D, HP=HP, bk=bk),
        grid=(B, HP),
        in_specs=[pair, pair],
        out_specs=pl.BlockSpec((R, L), lambda b, hp: (0, 0)),
        out_shape=jax.ShapeDtypeStruct((R, L), jnp.float32),
    )(q2, k2)

    idx_all = pl.pallas_call(
        functools.partial(_topk_kernel, L=L, R=R, u=u, u_pad=u_pad),
        in_specs=[pl.BlockSpec((R, L), lambda: (0, 0))],
        out_specs=pl.BlockSpec((R, u_pad), lambda: (0, 0)),
        out_shape=jax.ShapeDtypeStruct((R, u_pad), jnp.int32),
    )(m_all)

    return pl.pallas_call(
        functools.partial(_attend_kernel, L=L, D=D, HP=HP, u=u, u_pad=u_pad),
        grid=(B, HP),
        in_specs=[
            pl.BlockSpec((R, u_pad), lambda b, hp: (0, 0)),
            pair,
            pair,
            pair,
        ],
        out_specs=pl.BlockSpec((1, 2, L, D), lambda b, hp: (b, hp, 0, 0)),
        out_shape=jax.ShapeDtypeStruct((B, H, L, D), jnp.float32),
    )(idx_all, q2, k2, v2)


# speedup vs baseline: 1.0164x; 1.0164x over previous
import functools
import math

import jax
import jax.numpy as jnp
import numpy as np
from jax.experimental import pallas as pl
from jax.experimental.pallas import tpu as pltpu


def _nt_dot(a, b):
    return jax.lax.dot_general(
        a, b, (((1,), (1,)), ((), ())), preferred_element_type=jnp.float32
    )


def _tn_dot_exact(a, b):
    return jax.lax.dot_general(
        a, b, (((0,), (0,)), ((), ())),
        preferred_element_type=jnp.float32,
        precision=jax.lax.Precision.HIGHEST,
    )


def _fused_kernel(q_ref, k_ref, v_ref, out_ref, m_scr, idx_scr, *, L, D, HP, R, u, u_pad, bk):
    ph = pl.program_id(0)
    b = pl.program_id(1)
    hp = pl.program_id(2)
    r0 = 2 * (b * HP + hp)

    @pl.when(ph == 0)
    def _measure():
        for hh in range(2):
            cols = slice(hh * D, (hh + 1) * D)
            q_h = q_ref[0][:, cols]
            m_acc = jnp.full((8, L), -jnp.inf, dtype=jnp.float32)
            s_acc = jnp.zeros((8, L), jnp.float32)
            for jb in range(L // bk):
                k_chunk = k_ref[0, jb * bk:(jb + 1) * bk, cols]
                s_t = _nt_dot(k_chunk, q_h)
                for sb in range(bk // 8):
                    v = s_t[sb * 8:(sb + 1) * 8, :]
                    m_acc = jnp.maximum(m_acc, v)
                    s_acc = s_acc + v
            m_max = jnp.max(m_acc, axis=0, keepdims=True)
            row_sum = jnp.sum(s_acc, axis=0, keepdims=True)
            m_scr[pl.ds(r0 + hh, 1), :] = m_max - row_sum * (1.0 / L)

    @pl.when((ph == 1) & (b == 0) & (hp == 0))
    def _topk():
        m_cur = m_scr[...]
        idx2d = jax.lax.broadcasted_iota(jnp.int32, (R, L), 1)
        for t in range(u_pad):
            if t < u:
                rmax = jnp.max(m_cur, axis=1, keepdims=True)
                sel = jnp.min(
                    jnp.where(m_cur == rmax, idx2d, L), axis=1, keepdims=True
                )
                idx_scr[:, t:t + 1] = sel
                m_cur = jnp.where(idx2d == sel, -jnp.inf, m_cur)
            else:
                idx_scr[:, t:t + 1] = jnp.full((R, 1), -1, jnp.int32)

    @pl.when(ph == 1)
    def _attend():
        lane_i = jax.lax.broadcasted_iota(jnp.int32, (u_pad, L), 1)
        scale = 1.0 / math.sqrt(D)
        for hh in range(2):
            cols = slice(hh * D, (hh + 1) * D)
            idx_row = idx_scr[pl.ds(r0 + hh, 1), :]
            idx_col = jnp.transpose(idx_row, (1, 0))
            p_sel = jnp.where(idx_col == lane_i, 1.0, 0.0)
            q_h = q_ref[0][:, cols]
            k_h = k_ref[0][:, cols]
            v_h = v_ref[0][:, cols]
            q_r = jax.lax.dot_general(
                p_sel, q_h, (((1,), (0,)), ((), ())),
                preferred_element_type=jnp.float32,
                precision=jax.lax.Precision.HIGHEST,
            )
            scores = _nt_dot(q_r, k_h) * scale
            s_max = jnp.max(scores, axis=1, keepdims=True)
            s_exp = jnp.exp(scores - s_max)
            attn = s_exp / jnp.sum(s_exp, axis=1, keepdims=True)
            upd = jax.lax.dot_general(
                attn, v_h, (((1,), (0,)), ((), ())),
                preferred_element_type=jnp.float32,
            )
            scat = _tn_dot_exact(p_sel, upd)
            msk = _tn_dot_exact(p_sel, jnp.ones((u_pad, 1), jnp.float32))
            v_mean = jnp.sum(v_h, axis=0, keepdims=True) * (1.0 / L)
            out_ref[0, hh] = jnp.where(
                msk > 0.5, scat, jnp.broadcast_to(v_mean, (L, D))
            )


def kernel(queries, keys, values, attn_mask):
    B, L, H, D = queries.shape
    assert H % 2 == 0
    HP = H // 2
    R = B * H
    u = 5 * int(np.ceil(np.log(L)))
    u = min(u, L)
    u_pad = ((u + 7) // 8) * 8
    bk = min(256, L)

    q2 = jnp.reshape(queries, (B, L, H * D))
    k2 = jnp.reshape(keys, (B, L, H * D))
    v2 = jnp.reshape(values, (B, L, H * D))

    pair = pl.BlockSpec((1, L, 2 * D), lambda ph, b, hp: (b, 0, hp))
    v_spec = pl.BlockSpec(
        (1, L, 2 * D),
        lambda ph, b, hp: (jnp.where(ph == 0, 0, b), 0, jnp.where(ph == 0, 0, hp)),
    )
    out_spec = pl.BlockSpec(
        (1, 2, L, D),
        lambda ph, b, hp: (jnp.where(ph == 0, 0, b), jnp.where(ph == 0, 0, hp), 0, 0),
    )

    return pl.pallas_call(
        functools.partial(
            _fused_kernel, L=L, D=D, HP=HP, R=R, u=u, u_pad=u_pad, bk=bk
        ),
        grid=(2, B, HP),
        in_specs=[pair, pair, v_spec],
        out_specs=out_spec,
        out_shape=jax.ShapeDtypeStruct((B, H, L, D), jnp.float32),
        scratch_shapes=[
            pltpu.VMEM((R, L), jnp.float32),
            pltpu.VMEM((R, u_pad), jnp.int32),
        ],
    )(q2, k2, v2)


# speedup vs baseline: 1.0182x; 1.0018x over previous
import functools
import math

import jax
import jax.numpy as jnp
import numpy as np
from jax.experimental import pallas as pl
from jax.experimental.pallas import tpu as pltpu


def _nt_dot(a, b):
    return jax.lax.dot_general(
        a, b, (((1,), (1,)), ((), ())), preferred_element_type=jnp.float32
    )


def _tn_dot_exact(a, b):
    return jax.lax.dot_general(
        a, b, (((0,), (0,)), ((), ())),
        preferred_element_type=jnp.float32,
        precision=jax.lax.Precision.HIGHEST,
    )


def _fused_kernel(q_ref, k_ref, v_ref, out_ref, m_scr, idx_scr, *, L, D, HP, R, u, u_pad, bk):
    ph = pl.program_id(0)
    b = pl.program_id(1)
    hp = pl.program_id(2)
    r0 = 2 * (b * HP + hp)

    @pl.when(ph == 0)
    def _measure():
        for hh in range(2):
            cols = slice(hh * D, (hh + 1) * D)
            q_h = q_ref[0][:, cols]
            m_acc = jnp.full((8, L), -jnp.inf, dtype=jnp.float32)
            s_acc = jnp.zeros((8, L), jnp.float32)
            for jb in range(L // bk):
                k_chunk = k_ref[0, jb * bk:(jb + 1) * bk, cols]
                s_t = _nt_dot(k_chunk, q_h)
                for sb in range(bk // 8):
                    v = s_t[sb * 8:(sb + 1) * 8, :]
                    m_acc = jnp.maximum(m_acc, v)
                    s_acc = s_acc + v
            m_max = jnp.max(m_acc, axis=0, keepdims=True)
            row_sum = jnp.sum(s_acc, axis=0, keepdims=True)
            m_scr[pl.ds(r0 + hh, 1), :] = m_max - row_sum * (1.0 / L)

    @pl.when((ph == 1) & (b == 0) & (hp == 0))
    def _topk():
        m_cur = m_scr[...]
        idx2d = jax.lax.broadcasted_iota(jnp.int32, (R, L), 1)
        for t in range(u_pad):
            if t < u:
                rmax = jnp.max(m_cur, axis=1, keepdims=True)
                sel = jnp.min(
                    jnp.where(m_cur == rmax, idx2d, L), axis=1, keepdims=True
                )
                idx_scr[:, t:t + 1] = sel
                m_cur = jnp.where(idx2d == sel, -jnp.inf, m_cur)
            else:
                idx_scr[:, t:t + 1] = jnp.full((R, 1), -1, jnp.int32)

    @pl.when(ph == 1)
    def _attend():
        lane_i = jax.lax.broadcasted_iota(jnp.int32, (u_pad, L), 1)
        scale = 1.0 / math.sqrt(D)
        for hh in range(2):
            cols = slice(hh * D, (hh + 1) * D)
            idx_row = idx_scr[pl.ds(r0 + hh, 1), :]
            idx_col = jnp.transpose(idx_row, (1, 0))
            p_sel = jnp.where(idx_col == lane_i, 1.0, 0.0)
            q_h = q_ref[0][:, cols]
            k_h = k_ref[0][:, cols]
            v_h = v_ref[0][:, cols]
            q_r = jax.lax.dot_general(
                p_sel, q_h, (((1,), (0,)), ((), ())),
                preferred_element_type=jnp.float32,
                precision=jax.lax.Precision.HIGHEST,
            )
            scores = _nt_dot(q_r, k_h) * scale
            s_max = jnp.max(scores, axis=1, keepdims=True)
            s_exp = jnp.exp(scores - s_max)
            attn = s_exp / jnp.sum(s_exp, axis=1, keepdims=True)
            upd = jax.lax.dot_general(
                attn, v_h, (((1,), (0,)), ((), ())),
                preferred_element_type=jnp.float32,
            )
            scat = _tn_dot_exact(p_sel, upd)
            msk = _tn_dot_exact(p_sel, jnp.ones((u_pad, 1), jnp.float32))
            v_mean = jnp.sum(v_h, axis=0, keepdims=True) * (1.0 / L)
            out_ref[0, hh] = jnp.where(
                msk > 0.5, scat, jnp.broadcast_to(v_mean, (L, D))
            )


def kernel(queries, keys, values, attn_mask):
    B, L, H, D = queries.shape
    assert H % 2 == 0
    HP = H // 2
    R = B * H
    u = 5 * int(np.ceil(np.log(L)))
    u = min(u, L)
    u_pad = ((u + 7) // 8) * 8
    bk = min(512, L)

    q2 = jnp.reshape(queries, (B, L, H * D))
    k2 = jnp.reshape(keys, (B, L, H * D))
    v2 = jnp.reshape(values, (B, L, H * D))

    pair = pl.BlockSpec((1, L, 2 * D), lambda ph, b, hp: (b, 0, hp))
    v_spec = pl.BlockSpec(
        (1, L, 2 * D),
        lambda ph, b, hp: (jnp.where(ph == 0, 0, b), 0, jnp.where(ph == 0, 0, hp)),
    )
    out_spec = pl.BlockSpec(
        (1, 2, L, D),
        lambda ph, b, hp: (jnp.where(ph == 0, 0, b), jnp.where(ph == 0, 0, hp), 0, 0),
    )

    return pl.pallas_call(
        functools.partial(
            _fused_kernel, L=L, D=D, HP=HP, R=R, u=u, u_pad=u_pad, bk=bk
        ),
        grid=(2, B, HP),
        in_specs=[pair, pair, v_spec],
        out_specs=out_spec,
        out_shape=jax.ShapeDtypeStruct((B, H, L, D), jnp.float32),
        scratch_shapes=[
            pltpu.VMEM((R, L), jnp.float32),
            pltpu.VMEM((R, u_pad), jnp.int32),
        ],
    )(q2, k2, v2)
